# baseline (speedup 1.0000x reference)
import functools

import jax
import jax.numpy as jnp
from jax import lax
from jax.experimental import pallas as pl
from jax.experimental.pallas import tpu as pltpu
from jax.experimental.pallas import tpu_sc as plsc

_N1 = 25000
_N2 = 6250
_LANES = 16


def _mm_kernel(a_ref, b_ref, wa_ref, wb_ref, bias_ref, o_ref,
               *, a_transposed, b_transposed, out_transposed):
    wa = wa_ref[...]
    wb = wb_ref[...]
    if out_transposed:
        if a_transposed:
            acc = lax.dot_general(wa, a_ref[...], (((0,), (0,)), ((), ())),
                                  preferred_element_type=jnp.float32)
        else:
            acc = lax.dot_general(wa, a_ref[...], (((0,), (1,)), ((), ())),
                                  preferred_element_type=jnp.float32)
        if b_transposed:
            acc = acc + lax.dot_general(wb, b_ref[...], (((0,), (0,)), ((), ())),
                                        preferred_element_type=jnp.float32)
        else:
            acc = acc + lax.dot_general(wb, b_ref[...], (((0,), (1,)), ((), ())),
                                        preferred_element_type=jnp.float32)
        o_ref[...] = jnp.maximum(acc + bias_ref[...].reshape(-1, 1), 0.0)
    else:
        if a_transposed:
            acc = lax.dot_general(a_ref[...], wa, (((0,), (0,)), ((), ())),
                                  preferred_element_type=jnp.float32)
        else:
            acc = jnp.dot(a_ref[...], wa, preferred_element_type=jnp.float32)
        if b_transposed:
            acc = acc + lax.dot_general(b_ref[...], wb, (((0,), (0,)), ((), ())),
                                        preferred_element_type=jnp.float32)
        else:
            acc = acc + jnp.dot(b_ref[...], wb, preferred_element_type=jnp.float32)
        o_ref[...] = jnp.maximum(acc + bias_ref[...].reshape(1, -1), 0.0)


def _mm(a, b, wa, wb, bias, *, a_transposed=False, b_transposed=False,
        out_transposed=False, bn=2048):
    n = a.shape[1] if a_transposed else a.shape[0]
    grid = (pl.cdiv(n, bn),)

    def spec_for(arr, transposed):
        if transposed:
            return pl.BlockSpec((arr.shape[0], bn), lambda i: (0, i))
        return pl.BlockSpec((bn, arr.shape[1]), lambda i: (i, 0))

    dout = wa.shape[1]
    if out_transposed:
        out_spec = pl.BlockSpec((dout, bn), lambda i: (0, i))
        out_shape = jax.ShapeDtypeStruct((dout, n), jnp.float32)
    else:
        out_spec = pl.BlockSpec((bn, dout), lambda i: (i, 0))
        out_shape = jax.ShapeDtypeStruct((n, dout), jnp.float32)
    body = functools.partial(_mm_kernel, a_transposed=a_transposed,
                             b_transposed=b_transposed,
                             out_transposed=out_transposed)
    return pl.pallas_call(
        body,
        grid=grid,
        in_specs=[
            spec_for(a, a_transposed),
            spec_for(b, b_transposed),
            pl.BlockSpec(wa.shape, lambda i: (0, 0)),
            pl.BlockSpec(wb.shape, lambda i: (0, 0)),
            pl.BlockSpec((1, bias.shape[0]), lambda i: (0, 0)),
        ],
        out_specs=out_spec,
        out_shape=out_shape,
    )(a, b, wa, wb, bias.reshape(1, -1))


_CP = 2000


def _pool_max_body(ht_ref, ids_ref, out_ref, ids_v, hv, a0, a1, a2, a3,
                   *, np_, ns, out_transposed):
    accs = (a0, a1, a2, a3)
    cid = lax.axis_index("c")
    sid = lax.axis_index("s")
    wid = sid * 2 + cid
    f0 = wid * 4

    zeros = jnp.zeros((_LANES,), jnp.float32)
    nz = ns // _LANES

    def zero_body(i, carry):
        for acc in accs:
            acc[pl.ds(i * _LANES, _LANES)] = zeros
        return carry
    lax.fori_loop(0, nz, zero_body, 0)
    if ns % _LANES:
        for acc in accs:
            acc[pl.ds(ns - _LANES, _LANES)] = zeros

    nfull = np_ // _CP
    tail = np_ - nfull * _CP

    def do_groups(cp, dirty):
        ngrp = cp // _LANES

        def grp(g, d):
            c = ids_v[pl.ds(g * _LANES, _LANES)]
            for j in range(4):
                v = hv[j, pl.ds(g * _LANES, _LANES)]
                back = plsc.load_gather(accs[j], [c])
                pend = v > back
                plsc.store_scatter(accs[j], [c], v, mask=pend)
                d = jnp.logical_or(d, pend)
            return d
        dirty = lax.fori_loop(0, ngrp, grp, dirty)
        if cp % _LANES:
            c = ids_v[pl.ds(cp - _LANES, _LANES)]
            for j in range(4):
                v = hv[j, pl.ds(cp - _LANES, _LANES)]
                back = plsc.load_gather(accs[j], [c])
                pend = v > back
                plsc.store_scatter(accs[j], [c], v, mask=pend)
                dirty = jnp.logical_or(dirty, pend)
        return dirty

    def sweep(_):
        def chunk_body(ci, dirty):
            start = ci * _CP
            pltpu.sync_copy(ids_ref.at[pl.ds(start, _CP)], ids_v)
            pltpu.sync_copy(ht_ref.at[pl.ds(f0, 4), pl.ds(start, _CP)], hv)
            return do_groups(_CP, dirty)

        dirty = lax.fori_loop(0, nfull, chunk_body,
                              jnp.zeros((_LANES,), jnp.bool_))
        if tail:
            start = nfull * _CP
            pltpu.sync_copy(ids_ref.at[pl.ds(start, tail)], ids_v.at[pl.ds(0, tail)])
            pltpu.sync_copy(ht_ref.at[pl.ds(f0, 4), pl.ds(start, tail)],
                            hv.at[:, pl.ds(0, tail)])
            dirty = do_groups(tail, dirty)
        return jnp.any(dirty)

    lax.while_loop(lambda p: p, sweep, jnp.bool_(True))

    if out_transposed:
        for j in range(4):
            pltpu.sync_copy(accs[j], out_ref.at[f0 + j])
    else:
        for j in range(4):
            pltpu.sync_copy(accs[j], out_ref.at[:, f0 + j])


def _pool_max(ht, ids, ns, out_transposed):
    np_ = ids.shape[0]
    mesh = plsc.VectorSubcoreMesh(core_axis_name="c", subcore_axis_name="s")
    out_shape = (128, ns) if out_transposed else (ns, 128)
    body = functools.partial(_pool_max_body, np_=np_, ns=ns,
                             out_transposed=out_transposed)
    k = pl.kernel(
        body,
        out_type=jax.ShapeDtypeStruct(out_shape, jnp.float32),
        mesh=mesh,
        compiler_params=pltpu.CompilerParams(use_tc_tiling_on_sc=False,
                                             needs_layout_passes=False),
        scratch_types=[
            pltpu.VMEM((_CP,), jnp.int32),
            pltpu.VMEM((4, _CP), jnp.float32),
            pltpu.VMEM((ns,), jnp.float32),
            pltpu.VMEM((ns,), jnp.float32),
            pltpu.VMEM((ns,), jnp.float32),
            pltpu.VMEM((ns,), jnp.float32),
        ],
    )
    return k(ht, ids)


def kernel(x, pos, cluster0, cluster1, Wp0, bp0, Wp1, bp1, Wm1, bm1, Wm0, bm0):
    D = x.shape[1]
    c0 = cluster0.astype(jnp.int32)
    c1 = cluster1.astype(jnp.int32)

    h0t = _mm(x, pos, Wp0[:D], Wp0[D:], bp0, out_transposed=True)
    x1t = _pool_max(h0t, c0, _N1, out_transposed=True)
    cnt = jax.ops.segment_sum(jnp.ones((c0.shape[0], 1), jnp.float32),
                              c0, num_segments=_N1)
    pos1 = jax.ops.segment_sum(pos, c0, num_segments=_N1) / jnp.maximum(cnt, 1.0)

    h1t = _mm(x1t, pos1, Wp1[:D], Wp1[D:], bp1, a_transposed=True,
              out_transposed=True)
    x2t = _pool_max(h1t, c1, _N2, out_transposed=True)
    x2 = x2t.T

    up1 = jnp.take(x2, c1, axis=0)
    h2 = _mm(up1, x1t, Wm1[:D], Wm1[D:], bm1, b_transposed=True)
    up0 = jnp.take(h2, c0, axis=0)
    out = _mm(up0, x, Wm0[:D], Wm0[D:], bm0)
    return out


# speedup vs baseline: 1.1089x; 1.1089x over previous
import functools

import jax
import jax.numpy as jnp
from jax import lax
from jax.experimental import pallas as pl
from jax.experimental.pallas import tpu as pltpu
from jax.experimental.pallas import tpu_sc as plsc

_N1 = 25000
_N2 = 6250
_LANES = 16


def _mm_kernel(a_ref, b_ref, wa_ref, wb_ref, bias_ref, o_ref,
               *, a_transposed, b_transposed, out_transposed):
    wa = wa_ref[...]
    wb = wb_ref[...]
    if out_transposed:
        if a_transposed:
            acc = lax.dot_general(wa, a_ref[...], (((0,), (0,)), ((), ())),
                                  preferred_element_type=jnp.float32)
        else:
            acc = lax.dot_general(wa, a_ref[...], (((0,), (1,)), ((), ())),
                                  preferred_element_type=jnp.float32)
        if b_transposed:
            acc = acc + lax.dot_general(wb, b_ref[...], (((0,), (0,)), ((), ())),
                                        preferred_element_type=jnp.float32)
        else:
            acc = acc + lax.dot_general(wb, b_ref[...], (((0,), (1,)), ((), ())),
                                        preferred_element_type=jnp.float32)
        o_ref[...] = jnp.maximum(acc + bias_ref[...].reshape(-1, 1), 0.0)
    else:
        if a_transposed:
            acc = lax.dot_general(a_ref[...], wa, (((0,), (0,)), ((), ())),
                                  preferred_element_type=jnp.float32)
        else:
            acc = jnp.dot(a_ref[...], wa, preferred_element_type=jnp.float32)
        if b_transposed:
            acc = acc + lax.dot_general(b_ref[...], wb, (((0,), (0,)), ((), ())),
                                        preferred_element_type=jnp.float32)
        else:
            acc = acc + jnp.dot(b_ref[...], wb, preferred_element_type=jnp.float32)
        o_ref[...] = jnp.maximum(acc + bias_ref[...].reshape(1, -1), 0.0)


def _mm(a, b, wa, wb, bias, *, a_transposed=False, b_transposed=False,
        out_transposed=False, bn=2048):
    n = a.shape[1] if a_transposed else a.shape[0]
    grid = (pl.cdiv(n, bn),)

    def spec_for(arr, transposed):
        if transposed:
            return pl.BlockSpec((arr.shape[0], bn), lambda i: (0, i))
        return pl.BlockSpec((bn, arr.shape[1]), lambda i: (i, 0))

    dout = wa.shape[1]
    if out_transposed:
        out_spec = pl.BlockSpec((dout, bn), lambda i: (0, i))
        out_shape = jax.ShapeDtypeStruct((dout, n), jnp.float32)
    else:
        out_spec = pl.BlockSpec((bn, dout), lambda i: (i, 0))
        out_shape = jax.ShapeDtypeStruct((n, dout), jnp.float32)
    body = functools.partial(_mm_kernel, a_transposed=a_transposed,
                             b_transposed=b_transposed,
                             out_transposed=out_transposed)
    return pl.pallas_call(
        body,
        grid=grid,
        in_specs=[
            spec_for(a, a_transposed),
            spec_for(b, b_transposed),
            pl.BlockSpec(wa.shape, lambda i: (0, 0)),
            pl.BlockSpec(wb.shape, lambda i: (0, 0)),
            pl.BlockSpec((1, bias.shape[0]), lambda i: (0, 0)),
        ],
        out_specs=out_spec,
        out_shape=out_shape,
    )(a, b, wa, wb, bias.reshape(1, -1))


_CP = 2000


def _pool_max_body(ht_ref, ids_ref, out_ref, ids0, ids1, hv0, hv1,
                   a0, a1, a2, a3, s_i0, s_i1, s_h0, s_h1,
                   *, np_, ns, out_transposed):
    accs = (a0, a1, a2, a3)
    bufs = ((ids0, hv0, s_i0, s_h0), (ids1, hv1, s_i1, s_h1))
    cid = lax.axis_index("c")
    sid = lax.axis_index("s")
    wid = sid * 2 + cid
    f0 = wid * 4

    zeros = jnp.zeros((_LANES,), jnp.float32)
    nz = ns // _LANES

    def zero_body(i, carry):
        for acc in accs:
            acc[pl.ds(i * _LANES, _LANES)] = zeros
        return carry
    lax.fori_loop(0, nz, zero_body, 0)
    if ns % _LANES:
        for acc in accs:
            acc[pl.ds(ns - _LANES, _LANES)] = zeros

    nfull = np_ // _CP
    tail = np_ - nfull * _CP
    assert nfull % 2 == 0, "chunk count must be even for the 2-buffer ring"
    nclamp = nfull - 1

    def start_fetch(ci, b):
        start = ci * _CP
        ids_v, hv, s_i, s_h = bufs[b]
        pltpu.async_copy(ids_ref.at[pl.ds(start, _CP)], ids_v, s_i)
        pltpu.async_copy(ht_ref.at[pl.ds(f0, 4), pl.ds(start, _CP)], hv, s_h)

    def wait_fetch(b):
        ids_v, hv, s_i, s_h = bufs[b]
        pltpu.make_async_copy(ids_ref.at[pl.ds(0, _CP)], ids_v, s_i).wait()
        pltpu.make_async_copy(ht_ref.at[pl.ds(f0, 4), pl.ds(0, _CP)], hv,
                              s_h).wait()

    def do_groups(b, cp, dirty, base=0):
        ids_v, hv, _, _ = bufs[b]

        def one(off, d):
            c = ids_v[pl.ds(off, _LANES)]
            for j in range(4):
                v = hv[j, pl.ds(off, _LANES)]
                back = plsc.load_gather(accs[j], [c])
                pend = v > back
                plsc.store_scatter(accs[j], [c], v, mask=pend)
                d = jnp.logical_or(d, pend)
            return d

        ngrp = (cp - base) // _LANES
        unroll = 5 if ngrp % 5 == 0 else (4 if ngrp % 4 == 0 else 1)

        def grp(g, d):
            for u in range(unroll):
                d = one(base + (g * unroll + u) * _LANES, d)
            return d
        dirty = lax.fori_loop(0, ngrp // unroll, grp, dirty)
        if (cp - base) % _LANES:
            dirty = one(cp - _LANES, dirty)
        return dirty

    def sweep(_):
        start_fetch(0, 0)

        def pair_body(i, dirty):
            ci = i * 2
            start_fetch(ci + 1, 1)
            wait_fetch(0)
            dirty = do_groups(0, _CP, dirty)
            start_fetch(jnp.minimum(ci + 2, nclamp), 0)
            wait_fetch(1)
            dirty = do_groups(1, _CP, dirty)
            return dirty

        dirty = lax.fori_loop(0, nfull // 2, pair_body,
                              jnp.zeros((_LANES,), jnp.bool_))
        wait_fetch(0)
        dirty = do_groups(0, _CP, dirty)
        if tail:
            start = nfull * _CP
            ids_v, hv, s_i, s_h = bufs[1]
            pltpu.sync_copy(ids_ref.at[pl.ds(start, tail)],
                            ids_v.at[pl.ds(0, tail)])
            pltpu.sync_copy(ht_ref.at[pl.ds(f0, 4), pl.ds(start, tail)],
                            hv.at[:, pl.ds(0, tail)])
            dirty = do_groups(1, tail, dirty)
        return jnp.any(dirty)

    lax.while_loop(lambda p: p, sweep, jnp.bool_(True))

    if out_transposed:
        for j in range(4):
            pltpu.sync_copy(accs[j], out_ref.at[f0 + j])
    else:
        for j in range(4):
            pltpu.sync_copy(accs[j], out_ref.at[:, f0 + j])


def _pool_max(ht, ids, ns, out_transposed):
    np_ = ids.shape[0]
    mesh = plsc.VectorSubcoreMesh(core_axis_name="c", subcore_axis_name="s")
    out_shape = (128, ns) if out_transposed else (ns, 128)
    body = functools.partial(_pool_max_body, np_=np_, ns=ns,
                             out_transposed=out_transposed)
    k = pl.kernel(
        body,
        out_type=jax.ShapeDtypeStruct(out_shape, jnp.float32),
        mesh=mesh,
        compiler_params=pltpu.CompilerParams(use_tc_tiling_on_sc=False,
                                             needs_layout_passes=False),
        scratch_types=[
            pltpu.VMEM((_CP,), jnp.int32),
            pltpu.VMEM((_CP,), jnp.int32),
            pltpu.VMEM((4, _CP), jnp.float32),
            pltpu.VMEM((4, _CP), jnp.float32),
            pltpu.VMEM((ns,), jnp.float32),
            pltpu.VMEM((ns,), jnp.float32),
            pltpu.VMEM((ns,), jnp.float32),
            pltpu.VMEM((ns,), jnp.float32),
            pltpu.SemaphoreType.DMA,
            pltpu.SemaphoreType.DMA,
            pltpu.SemaphoreType.DMA,
            pltpu.SemaphoreType.DMA,
        ],
    )
    return k(ht, ids)


def kernel(x, pos, cluster0, cluster1, Wp0, bp0, Wp1, bp1, Wm1, bm1, Wm0, bm0):
    D = x.shape[1]
    c0 = cluster0.astype(jnp.int32)
    c1 = cluster1.astype(jnp.int32)

    h0t = _mm(x, pos, Wp0[:D], Wp0[D:], bp0, out_transposed=True)
    x1t = _pool_max(h0t, c0, _N1, out_transposed=True)
    cnt = jax.ops.segment_sum(jnp.ones((c0.shape[0], 1), jnp.float32),
                              c0, num_segments=_N1)
    pos1 = jax.ops.segment_sum(pos, c0, num_segments=_N1) / jnp.maximum(cnt, 1.0)

    h1t = _mm(x1t, pos1, Wp1[:D], Wp1[D:], bp1, a_transposed=True,
              out_transposed=True)
    x2t = _pool_max(h1t, c1, _N2, out_transposed=True)
    x2 = x2t.T

    up1 = jnp.take(x2, c1, axis=0)
    h2 = _mm(up1, x1t, Wm1[:D], Wm1[D:], bm1, b_transposed=True)
    up0 = jnp.take(h2, c0, axis=0)
    out = _mm(up0, x, Wm0[:D], Wm0[D:], bm0)
    return out


# speedup vs baseline: 1.4258x; 1.2858x over previous
import functools

import jax
import jax.numpy as jnp
from jax import lax
from jax.experimental import pallas as pl
from jax.experimental.pallas import tpu as pltpu
from jax.experimental.pallas import tpu_sc as plsc

_N1 = 25000
_N2 = 6250
_LANES = 16
_CP = 2000


def _sc_params():
    return pltpu.CompilerParams(use_tc_tiling_on_sc=False,
                                needs_layout_passes=False)


def _mm_kernel(a_ref, b_ref, wa_ref, wb_ref, bias_ref, o_ref,
               *, a_transposed, b_transposed, out_transposed):
    wa = wa_ref[...]
    wb = wb_ref[...]
    if out_transposed:
        ca = (((0,), (0 if a_transposed else 1,)), ((), ()))
        acc = lax.dot_general(wa, a_ref[...], ca,
                              preferred_element_type=jnp.float32)
        cb = (((0,), (0 if b_transposed else 1,)), ((), ()))
        acc = acc + lax.dot_general(wb, b_ref[...], cb,
                                    preferred_element_type=jnp.float32)
        o_ref[...] = jnp.maximum(acc + bias_ref[...].reshape(-1, 1), 0.0)
    else:
        if a_transposed:
            acc = lax.dot_general(a_ref[...], wa, (((0,), (0,)), ((), ())),
                                  preferred_element_type=jnp.float32)
        else:
            acc = jnp.dot(a_ref[...], wa, preferred_element_type=jnp.float32)
        if b_transposed:
            acc = acc + lax.dot_general(b_ref[...], wb, (((0,), (0,)), ((), ())),
                                        preferred_element_type=jnp.float32)
        else:
            acc = acc + jnp.dot(b_ref[...], wb, preferred_element_type=jnp.float32)
        o_ref[...] = jnp.maximum(acc + bias_ref[...].reshape(1, -1), 0.0)


def _mm(a, b, wa, wb, bias, *, a_transposed=False, b_transposed=False,
        out_transposed=False, bn=2048):
    n = a.shape[1] if a_transposed else a.shape[0]
    grid = (pl.cdiv(n, bn),)

    def spec_for(arr, transposed):
        if transposed:
            return pl.BlockSpec((arr.shape[0], bn), lambda i: (0, i))
        return pl.BlockSpec((bn, arr.shape[1]), lambda i: (i, 0))

    dout = wa.shape[1]
    if out_transposed:
        out_spec = pl.BlockSpec((dout, bn), lambda i: (0, i))
        out_shape = jax.ShapeDtypeStruct((dout, n), jnp.float32)
    else:
        out_spec = pl.BlockSpec((bn, dout), lambda i: (i, 0))
        out_shape = jax.ShapeDtypeStruct((n, dout), jnp.float32)
    body = functools.partial(_mm_kernel, a_transposed=a_transposed,
                             b_transposed=b_transposed,
                             out_transposed=out_transposed)
    return pl.pallas_call(
        body,
        grid=grid,
        in_specs=[
            spec_for(a, a_transposed),
            spec_for(b, b_transposed),
            pl.BlockSpec(wa.shape, lambda i: (0, 0)),
            pl.BlockSpec(wb.shape, lambda i: (0, 0)),
            pl.BlockSpec((1, bias.shape[0]), lambda i: (0, 0)),
        ],
        out_specs=out_spec,
        out_shape=out_shape,
    )(a, b, wa, wb, bias.reshape(1, -1))


def _mm2_kernel(x1t_ref, padd_ref, wx_ref, wp_ref, bias_ref, o_ref):
    s = jnp.sum(padd_ref[...], axis=1)
    cnt = jnp.maximum(s[3:4, :], 1.0)
    pos1 = s[:3, :] / cnt
    acc = lax.dot_general(wx_ref[...], x1t_ref[...], (((0,), (0,)), ((), ())),
                          preferred_element_type=jnp.float32)
    acc = acc + lax.dot_general(wp_ref[...], pos1, (((0,), (0,)), ((), ())),
                                preferred_element_type=jnp.float32)
    o_ref[...] = jnp.maximum(acc + bias_ref[...].reshape(-1, 1), 0.0)


def _mm2(x1t, padd, wx, wp, bias, bn=2048):
    n = x1t.shape[1]
    grid = (pl.cdiv(n, bn),)
    return pl.pallas_call(
        _mm2_kernel,
        grid=grid,
        in_specs=[
            pl.BlockSpec((128, bn), lambda i: (0, i)),
            pl.BlockSpec((4, 8, bn), lambda i: (0, 0, i)),
            pl.BlockSpec(wx.shape, lambda i: (0, 0)),
            pl.BlockSpec(wp.shape, lambda i: (0, 0)),
            pl.BlockSpec((1, bias.shape[0]), lambda i: (0, 0)),
        ],
        out_specs=pl.BlockSpec((128, bn), lambda i: (0, i)),
        out_shape=jax.ShapeDtypeStruct((128, n), jnp.float32),
    )(x1t, padd, wx, wp, bias.reshape(1, -1))


def _pool_max_body(ht_ref, ids_ref, out_ref, ids0, ids1, hv0, hv1,
                   a0, a1, a2, a3, s_i0, s_i1, s_h0, s_h1,
                   *, np_, ns, gather_back):
    accs = (a0, a1, a2, a3)
    bufs = ((ids0, hv0, s_i0, s_h0), (ids1, hv1, s_i1, s_h1))
    cid = lax.axis_index("c")
    sid = lax.axis_index("s")
    wid = sid * 2 + cid
    f0 = wid * 4

    zeros = jnp.zeros((_LANES,), jnp.float32)

    def zero_body(i, carry):
        for acc in accs:
            acc[pl.ds(i * _LANES, _LANES)] = zeros
        return carry
    lax.fori_loop(0, ns // _LANES, zero_body, 0)
    if ns % _LANES:
        for acc in accs:
            acc[pl.ds(ns - _LANES, _LANES)] = zeros

    nfull = np_ // _CP
    tail = np_ - nfull * _CP
    assert nfull % 2 == 0, "chunk count must be even for the 2-buffer ring"
    nclamp = nfull - 1

    def start_fetch(ci, b):
        start = ci * _CP
        ids_v, hv, s_i, s_h = bufs[b]
        pltpu.async_copy(ids_ref.at[pl.ds(start, _CP)], ids_v, s_i)
        pltpu.async_copy(ht_ref.at[pl.ds(f0, 4), pl.ds(start, _CP)], hv, s_h)

    def wait_fetch(b):
        ids_v, hv, s_i, s_h = bufs[b]
        pltpu.make_async_copy(ids_ref.at[pl.ds(0, _CP)], ids_v, s_i).wait()
        pltpu.make_async_copy(ht_ref.at[pl.ds(f0, 4), pl.ds(0, _CP)], hv,
                              s_h).wait()

    def do_groups(b, cp, dirty):
        ids_v, hv, _, _ = bufs[b]

        def one(off, d):
            c = ids_v[pl.ds(off, _LANES)]
            for j in range(4):
                v = hv[j, pl.ds(off, _LANES)]
                back = plsc.load_gather(accs[j], [c])
                pend = v > back
                plsc.store_scatter(accs[j], [c], v, mask=pend)
                d = jnp.logical_or(d, pend)
            return d

        ngrp = cp // _LANES
        unroll = 5 if ngrp % 5 == 0 else (4 if ngrp % 4 == 0 else 1)

        def grp(g, d):
            for u in range(unroll):
                d = one((g * unroll + u) * _LANES, d)
            return d
        dirty = lax.fori_loop(0, ngrp // unroll, grp, dirty)
        if cp % _LANES:
            dirty = one(cp - _LANES, dirty)
        return dirty

    def sweep(_):
        start_fetch(0, 0)

        def pair_body(i, dirty):
            ci = i * 2
            start_fetch(ci + 1, 1)
            wait_fetch(0)
            dirty = do_groups(0, _CP, dirty)
            start_fetch(jnp.minimum(ci + 2, nclamp), 0)
            wait_fetch(1)
            dirty = do_groups(1, _CP, dirty)
            return dirty

        dirty = lax.fori_loop(0, nfull // 2, pair_body,
                              jnp.zeros((_LANES,), jnp.bool_))
        wait_fetch(0)
        dirty = do_groups(0, _CP, dirty)
        if tail:
            ids_v, hv, s_i, s_h = bufs[1]
            start = nfull * _CP
            pltpu.sync_copy(ids_ref.at[pl.ds(start, tail)],
                            ids_v.at[pl.ds(0, tail)])
            pltpu.sync_copy(ht_ref.at[pl.ds(f0, 4), pl.ds(start, tail)],
                            hv.at[:, pl.ds(0, tail)])
            dirty = do_groups(1, tail, dirty)
        return jnp.any(dirty)

    lax.while_loop(lambda p: p, sweep, jnp.bool_(True))

    if not gather_back:
        for j in range(4):
            pltpu.sync_copy(accs[j], out_ref.at[f0 + j])
        return

    def compute_into(b, cp):
        ids_v, hv, _, _ = bufs[b]

        def one(off):
            c = ids_v[pl.ds(off, _LANES)]
            for j in range(4):
                hv[j, pl.ds(off, _LANES)] = plsc.load_gather(accs[j], [c])

        def grp(g, carry):
            one(g * _LANES)
            return carry
        lax.fori_loop(0, cp // _LANES, grp, 0)
        if cp % _LANES:
            one(cp - _LANES)

    def start_ids(ci, b):
        ids_v, _, s_i, _ = bufs[b]
        pltpu.async_copy(ids_ref.at[pl.ds(ci * _CP, _CP)], ids_v, s_i)

    def wait_ids(b):
        ids_v, _, s_i, _ = bufs[b]
        pltpu.make_async_copy(ids_ref.at[pl.ds(0, _CP)], ids_v, s_i).wait()

    def start_out(ci, b):
        _, hv, _, s_h = bufs[b]
        pltpu.async_copy(hv, out_ref.at[pl.ds(f0, 4), pl.ds(ci * _CP, _CP)],
                         s_h)

    def wait_out(b):
        _, hv, _, s_h = bufs[b]
        pltpu.make_async_copy(hv, out_ref.at[pl.ds(f0, 4), pl.ds(0, _CP)],
                              s_h).wait()

    start_ids(0, 0)

    def gpair(i, carry):
        ci = i * 2
        start_ids(ci + 1, 1)
        wait_ids(0)

        @pl.when(i > 0)
        def _():
            wait_out(0)
        compute_into(0, _CP)
        start_out(ci, 0)
        start_ids(jnp.minimum(ci + 2, nclamp), 0)
        wait_ids(1)

        @pl.when(i > 0)
        def _():
            wait_out(1)
        compute_into(1, _CP)
        start_out(ci + 1, 1)
        return carry
    lax.fori_loop(0, nfull // 2, gpair, 0)
    wait_ids(0)
    wait_out(0)
    wait_out(1)
    if tail:
        ids_v, hv, s_i, s_h = bufs[0]
        start = nfull * _CP
        pltpu.sync_copy(ids_ref.at[pl.ds(start, tail)],
                        ids_v.at[pl.ds(0, tail)])
        compute_into(0, tail)
        pltpu.sync_copy(hv.at[:, pl.ds(0, tail)],
                        out_ref.at[pl.ds(f0, 4), pl.ds(start, tail)])


def _pool_max(ht, ids, ns, *, gather_back):
    np_ = ids.shape[0]
    mesh = plsc.VectorSubcoreMesh(core_axis_name="c", subcore_axis_name="s")
    out_shape = (128, np_) if gather_back else (128, ns)
    body = functools.partial(_pool_max_body, np_=np_, ns=ns,
                             gather_back=gather_back)
    k = pl.kernel(
        body,
        out_type=jax.ShapeDtypeStruct(out_shape, jnp.float32),
        mesh=mesh,
        compiler_params=_sc_params(),
        scratch_types=[
            pltpu.VMEM((_CP,), jnp.int32),
            pltpu.VMEM((_CP,), jnp.int32),
            pltpu.VMEM((4, _CP), jnp.float32),
            pltpu.VMEM((4, _CP), jnp.float32),
            pltpu.VMEM((ns,), jnp.float32),
            pltpu.VMEM((ns,), jnp.float32),
            pltpu.VMEM((ns,), jnp.float32),
            pltpu.VMEM((ns,), jnp.float32),
            pltpu.SemaphoreType.DMA,
            pltpu.SemaphoreType.DMA,
            pltpu.SemaphoreType.DMA,
            pltpu.SemaphoreType.DMA,
        ],
    )
    return k(ht, ids)


def _pool_adds_body(ids_ref, pos_ref, out_ref, ids_v, posb, acc, *, np_, ns):
    cid = lax.axis_index("c")
    sid = lax.axis_index("s")
    wid = sid * 2 + cid
    col = wid % 4
    shard = wid // 4

    zeros = jnp.zeros((_LANES,), jnp.float32)

    def zero_body(i, carry):
        acc[pl.ds(i * _LANES, _LANES)] = zeros
        return carry
    lax.fori_loop(0, ns // _LANES, zero_body, 0)
    if ns % _LANES:
        acc[pl.ds(ns - _LANES, _LANES)] = zeros

    nchunks = np_ // _CP
    assert nchunks * _CP == np_
    iota = lax.iota(jnp.int32, _LANES)
    colsplat = jnp.full((_LANES,), 0, jnp.int32) + jnp.minimum(col, 2)
    is_cnt = col == 3
    ones = jnp.ones((_LANES,), jnp.float32)

    for k in range((nchunks + 7) // 8):
        ci = shard + 8 * k

        @pl.when(ci < nchunks)
        def _():
            start = ci * _CP
            pltpu.sync_copy(ids_ref.at[pl.ds(start, _CP)], ids_v)
            pltpu.sync_copy(pos_ref.at[pl.ds(start, _CP)], posb)

            def grp(g, carry):
                c = ids_v[pl.ds(g * _LANES, _LANES)]
                rows = g * _LANES + iota
                v = plsc.load_gather(posb, [rows, colsplat])
                v = jnp.where(is_cnt, ones, v)
                plsc.addupdate_scatter(acc, [c], v)
                return carry
            lax.fori_loop(0, _CP // _LANES, grp, 0)

    pltpu.sync_copy(acc, out_ref.at[col, shard])


def _pool_adds(ids, pos, ns):
    np_ = ids.shape[0]
    mesh = plsc.VectorSubcoreMesh(core_axis_name="c", subcore_axis_name="s")
    body = functools.partial(_pool_adds_body, np_=np_, ns=ns)
    k = pl.kernel(
        body,
        out_type=jax.ShapeDtypeStruct((4, 8, ns), jnp.float32),
        mesh=mesh,
        compiler_params=_sc_params(),
        scratch_types=[
            pltpu.VMEM((_CP,), jnp.int32),
            pltpu.VMEM((_CP, 3), jnp.float32),
            pltpu.VMEM((ns,), jnp.float32),
        ],
    )
    return k(ids, pos)


def _gather_cols_body(tab_ref, ids_ref, out_ref, tabv, ids0, ids1, ub0, ub1,
                      s_i0, s_i1, s_h0, s_h1, *, np_, ns):
    cid = lax.axis_index("c")
    sid = lax.axis_index("s")
    wid = sid * 2 + cid
    f0 = wid * 4
    bufs = ((ids0, ub0, s_i0, s_h0), (ids1, ub1, s_i1, s_h1))

    pltpu.sync_copy(tab_ref.at[pl.ds(f0, 4), :], tabv)

    nfull = np_ // _CP
    tail = np_ - nfull * _CP
    assert nfull % 2 == 0
    nclamp = nfull - 1

    def compute_into(b, cp):
        ids_v, ub, _, _ = bufs[b]

        def one(off):
            c = ids_v[pl.ds(off, _LANES)]
            for j in range(4):
                ub[j, pl.ds(off, _LANES)] = plsc.load_gather(tabv.at[j], [c])

        def grp(g, carry):
            one(g * _LANES)
            return carry
        lax.fori_loop(0, cp // _LANES, grp, 0)
        if cp % _LANES:
            one(cp - _LANES)

    def start_ids(ci, b):
        ids_v, _, s_i, _ = bufs[b]
        pltpu.async_copy(ids_ref.at[pl.ds(ci * _CP, _CP)], ids_v, s_i)

    def wait_ids(b):
        ids_v, _, s_i, _ = bufs[b]
        pltpu.make_async_copy(ids_ref.at[pl.ds(0, _CP)], ids_v, s_i).wait()

    def start_out(ci, b):
        _, ub, _, s_h = bufs[b]
        pltpu.async_copy(ub, out_ref.at[pl.ds(f0, 4), pl.ds(ci * _CP, _CP)],
                         s_h)

    def wait_out(b):
        _, ub, _, s_h = bufs[b]
        pltpu.make_async_copy(ub, out_ref.at[pl.ds(f0, 4), pl.ds(0, _CP)],
                              s_h).wait()

    start_ids(0, 0)

    def gpair(i, carry):
        ci = i * 2
        start_ids(ci + 1, 1)
        wait_ids(0)

        @pl.when(i > 0)
        def _():
            wait_out(0)
        compute_into(0, _CP)
        start_out(ci, 0)
        start_ids(jnp.minimum(ci + 2, nclamp), 0)
        wait_ids(1)

        @pl.when(i > 0)
        def _():
            wait_out(1)
        compute_into(1, _CP)
        start_out(ci + 1, 1)
        return carry
    lax.fori_loop(0, nfull // 2, gpair, 0)
    wait_ids(0)
    wait_out(0)
    wait_out(1)
    if tail:
        ids_v, ub, s_i, s_h = bufs[0]
        start = nfull * _CP
        pltpu.sync_copy(ids_ref.at[pl.ds(start, tail)],
                        ids_v.at[pl.ds(0, tail)])
        compute_into(0, tail)
        pltpu.sync_copy(ub.at[:, pl.ds(0, tail)],
                        out_ref.at[pl.ds(f0, 4), pl.ds(start, tail)])


def _gather_cols(tab_t, ids):
    np_ = ids.shape[0]
    ns = tab_t.shape[1]
    mesh = plsc.VectorSubcoreMesh(core_axis_name="c", subcore_axis_name="s")
    body = functools.partial(_gather_cols_body, np_=np_, ns=ns)
    k = pl.kernel(
        body,
        out_type=jax.ShapeDtypeStruct((128, np_), jnp.float32),
        mesh=mesh,
        compiler_params=_sc_params(),
        scratch_types=[
            pltpu.VMEM((4, ns), jnp.float32),
            pltpu.VMEM((_CP,), jnp.int32),
            pltpu.VMEM((_CP,), jnp.int32),
            pltpu.VMEM((4, _CP), jnp.float32),
            pltpu.VMEM((4, _CP), jnp.float32),
            pltpu.SemaphoreType.DMA,
            pltpu.SemaphoreType.DMA,
            pltpu.SemaphoreType.DMA,
            pltpu.SemaphoreType.DMA,
        ],
    )
    return k(tab_t, ids)


def kernel(x, pos, cluster0, cluster1, Wp0, bp0, Wp1, bp1, Wm1, bm1, Wm0, bm0):
    D = x.shape[1]
    c0 = cluster0.astype(jnp.int32)
    c1 = cluster1.astype(jnp.int32)

    h0t = _mm(x, pos, Wp0[:D], Wp0[D:], bp0, out_transposed=True)
    x1t = _pool_max(h0t, c0, _N1, gather_back=False)
    padd = _pool_adds(c0, pos, _N1)

    h1t = _mm2(x1t, padd, Wp1[:D], Wp1[D:], bp1)
    up1t = _pool_max(h1t, c1, _N2, gather_back=True)

    h2t = _mm(up1t, x1t, Wm1[:D], Wm1[D:], bm1, a_transposed=True,
              b_transposed=True, out_transposed=True)
    up0t = _gather_cols(h2t, c0)
    out = _mm(up0t, x, Wm0[:D], Wm0[D:], bm0, a_transposed=True)
    return out


# speedup vs baseline: 2.9030x; 2.0361x over previous
import functools

import jax
import jax.numpy as jnp
from jax import lax
from jax.experimental import pallas as pl
from jax.experimental.pallas import tpu as pltpu
from jax.experimental.pallas import tpu_sc as plsc

_N1 = 25000
_N2 = 6250
_LANES = 16
_CP = 2000


def _sc_params():
    return pltpu.CompilerParams(use_tc_tiling_on_sc=False,
                                needs_layout_passes=False)


def _mm_kernel(a_ref, b_ref, wa_ref, wb_ref, bias_ref, o_ref,
               *, a_transposed, b_transposed, out_transposed):
    wa = wa_ref[...]
    wb = wb_ref[...]
    if out_transposed:
        ca = (((0,), (0 if a_transposed else 1,)), ((), ()))
        acc = lax.dot_general(wa, a_ref[...], ca,
                              preferred_element_type=jnp.float32)
        cb = (((0,), (0 if b_transposed else 1,)), ((), ()))
        acc = acc + lax.dot_general(wb, b_ref[...], cb,
                                    preferred_element_type=jnp.float32)
        o_ref[...] = jnp.maximum(acc + bias_ref[...].reshape(-1, 1), 0.0)
    else:
        if a_transposed:
            acc = lax.dot_general(a_ref[...], wa, (((0,), (0,)), ((), ())),
                                  preferred_element_type=jnp.float32)
        else:
            acc = jnp.dot(a_ref[...], wa, preferred_element_type=jnp.float32)
        if b_transposed:
            acc = acc + lax.dot_general(b_ref[...], wb, (((0,), (0,)), ((), ())),
                                        preferred_element_type=jnp.float32)
        else:
            acc = acc + jnp.dot(b_ref[...], wb, preferred_element_type=jnp.float32)
        o_ref[...] = jnp.maximum(acc + bias_ref[...].reshape(1, -1), 0.0)


def _mm(a, b, wa, wb, bias, *, a_transposed=False, b_transposed=False,
        out_transposed=False, bn=2048):
    n = a.shape[1] if a_transposed else a.shape[0]
    grid = (pl.cdiv(n, bn),)

    def spec_for(arr, transposed):
        if transposed:
            return pl.BlockSpec((arr.shape[0], bn), lambda i: (0, i))
        return pl.BlockSpec((bn, arr.shape[1]), lambda i: (i, 0))

    dout = wa.shape[1]
    if out_transposed:
        out_spec = pl.BlockSpec((dout, bn), lambda i: (0, i))
        out_shape = jax.ShapeDtypeStruct((dout, n), jnp.float32)
    else:
        out_spec = pl.BlockSpec((bn, dout), lambda i: (i, 0))
        out_shape = jax.ShapeDtypeStruct((n, dout), jnp.float32)
    body = functools.partial(_mm_kernel, a_transposed=a_transposed,
                             b_transposed=b_transposed,
                             out_transposed=out_transposed)
    return pl.pallas_call(
        body,
        grid=grid,
        in_specs=[
            spec_for(a, a_transposed),
            spec_for(b, b_transposed),
            pl.BlockSpec(wa.shape, lambda i: (0, 0)),
            pl.BlockSpec(wb.shape, lambda i: (0, 0)),
            pl.BlockSpec((1, bias.shape[0]), lambda i: (0, 0)),
        ],
        out_specs=out_spec,
        out_shape=out_shape,
    )(a, b, wa, wb, bias.reshape(1, -1))


def _mm2_kernel(x1t_ref, padd_ref, wx_ref, wp_ref, bias_ref, o_ref):
    s = jnp.sum(padd_ref[...], axis=1)
    cnt = jnp.maximum(s[3:4, :], 1.0)
    pos1 = s[:3, :] / cnt
    acc = lax.dot_general(wx_ref[...], x1t_ref[...], (((0,), (0,)), ((), ())),
                          preferred_element_type=jnp.float32)
    acc = acc + lax.dot_general(wp_ref[...], pos1, (((0,), (0,)), ((), ())),
                                preferred_element_type=jnp.float32)
    o_ref[...] = jnp.maximum(acc + bias_ref[...].reshape(-1, 1), 0.0)


def _mm2(x1t, padd, wx, wp, bias, bn=2048):
    n = x1t.shape[1]
    grid = (pl.cdiv(n, bn),)
    return pl.pallas_call(
        _mm2_kernel,
        grid=grid,
        in_specs=[
            pl.BlockSpec((128, bn), lambda i: (0, i)),
            pl.BlockSpec((4, 8, bn), lambda i: (0, 0, i)),
            pl.BlockSpec(wx.shape, lambda i: (0, 0)),
            pl.BlockSpec(wp.shape, lambda i: (0, 0)),
            pl.BlockSpec((1, bias.shape[0]), lambda i: (0, 0)),
        ],
        out_specs=pl.BlockSpec((128, bn), lambda i: (0, i)),
        out_shape=jax.ShapeDtypeStruct((128, n), jnp.float32),
    )(x1t, padd, wx, wp, bias.reshape(1, -1))


def _pool_max_body(ht_ref, ids_ref, out_ref, ids0, ids1, hv0, hv1,
                   a0, a1, a2, a3, s_i0, s_i1, s_h0, s_h1,
                   *, np_, ns, gather_back):
    accs = (a0, a1, a2, a3)
    bufs = ((ids0, hv0, s_i0, s_h0), (ids1, hv1, s_i1, s_h1))
    cid = lax.axis_index("c")
    sid = lax.axis_index("s")
    wid = sid * 2 + cid
    f0 = wid * 4

    zeros = jnp.zeros((_LANES,), jnp.float32)

    def zero_body(i, carry):
        for acc in accs:
            acc[pl.ds(i * _LANES, _LANES)] = zeros
        return carry
    lax.fori_loop(0, ns // _LANES, zero_body, 0)
    if ns % _LANES:
        for acc in accs:
            acc[pl.ds(ns - _LANES, _LANES)] = zeros

    nfull = np_ // _CP
    tail = np_ - nfull * _CP
    assert nfull % 2 == 0, "chunk count must be even for the 2-buffer ring"
    nclamp = nfull - 1

    def start_fetch(ci, b):
        start = ci * _CP
        ids_v, hv, s_i, s_h = bufs[b]
        pltpu.async_copy(ids_ref.at[pl.ds(start, _CP)], ids_v, s_i)
        pltpu.async_copy(ht_ref.at[pl.ds(f0, 4), pl.ds(start, _CP)], hv, s_h)

    def wait_fetch(b):
        ids_v, hv, s_i, s_h = bufs[b]
        pltpu.make_async_copy(ids_ref.at[pl.ds(0, _CP)], ids_v, s_i).wait()
        pltpu.make_async_copy(ht_ref.at[pl.ds(f0, 4), pl.ds(0, _CP)], hv,
                              s_h).wait()

    def do_groups(b, cp, dirty, racy):
        ids_v, hv, _, _ = bufs[b]

        def one(off, d):
            c = ids_v[pl.ds(off, _LANES)]
            for j in range(4):
                v = hv[j, pl.ds(off, _LANES)]
                back = plsc.load_gather(accs[j], [c])
                pend = v > back
                plsc.store_scatter(accs[j], [c], v, mask=pend)
                d = jnp.logical_or(d, pend)
            return d

        ngrp = cp // _LANES
        if racy:
            @plsc.parallel_loop(0, ngrp, 1, unroll=4, carry=dirty)
            def ploop(g, d):
                return one(g * _LANES, d)
            dirty = ploop
        else:
            unroll = 5 if ngrp % 5 == 0 else (4 if ngrp % 4 == 0 else 1)

            def grp(g, d):
                for u in range(unroll):
                    d = one((g * unroll + u) * _LANES, d)
                return d
            dirty = lax.fori_loop(0, ngrp // unroll, grp, dirty)
        if cp % _LANES:
            dirty = one(cp - _LANES, dirty)
        return dirty

    def sweep(racy):
        start_fetch(0, 0)

        def pair_body(i, dirty):
            ci = i * 2
            start_fetch(ci + 1, 1)
            wait_fetch(0)
            dirty = do_groups(0, _CP, dirty, racy)
            start_fetch(jnp.minimum(ci + 2, nclamp), 0)
            wait_fetch(1)
            dirty = do_groups(1, _CP, dirty, racy)
            return dirty

        dirty = lax.fori_loop(0, nfull // 2, pair_body,
                              jnp.zeros((_LANES,), jnp.bool_))
        wait_fetch(0)
        dirty = do_groups(0, _CP, dirty, racy)
        if tail:
            ids_v, hv, s_i, s_h = bufs[1]
            start = nfull * _CP
            pltpu.sync_copy(ids_ref.at[pl.ds(start, tail)],
                            ids_v.at[pl.ds(0, tail)])
            pltpu.sync_copy(ht_ref.at[pl.ds(f0, 4), pl.ds(start, tail)],
                            hv.at[:, pl.ds(0, tail)])
            dirty = do_groups(1, tail, dirty, racy)
        return jnp.any(dirty)

    def racy_phase(carry):
        _, it = carry
        return sweep(racy=True), it + 1

    dirty, _ = lax.while_loop(lambda c: jnp.logical_and(c[0], c[1] < 6),
                              racy_phase, (jnp.bool_(True), jnp.int32(0)))
    lax.while_loop(lambda p: p, lambda p: sweep(racy=False), dirty)

    if not gather_back:
        for j in range(4):
            pltpu.sync_copy(accs[j], out_ref.at[f0 + j])
        return

    def compute_into(b, cp):
        ids_v, hv, _, _ = bufs[b]

        def one(off):
            c = ids_v[pl.ds(off, _LANES)]
            for j in range(4):
                hv[j, pl.ds(off, _LANES)] = plsc.load_gather(accs[j], [c])

        @plsc.parallel_loop(0, cp // _LANES, 1, unroll=4)
        def ploop(g):
            one(g * _LANES)
        if cp % _LANES:
            one(cp - _LANES)

    def start_ids(ci, b):
        ids_v, _, s_i, _ = bufs[b]
        pltpu.async_copy(ids_ref.at[pl.ds(ci * _CP, _CP)], ids_v, s_i)

    def wait_ids(b):
        ids_v, _, s_i, _ = bufs[b]
        pltpu.make_async_copy(ids_ref.at[pl.ds(0, _CP)], ids_v, s_i).wait()

    def start_out(ci, b):
        _, hv, _, s_h = bufs[b]
        pltpu.async_copy(hv, out_ref.at[pl.ds(f0, 4), pl.ds(ci * _CP, _CP)],
                         s_h)

    def wait_out(b):
        _, hv, _, s_h = bufs[b]
        pltpu.make_async_copy(hv, out_ref.at[pl.ds(f0, 4), pl.ds(0, _CP)],
                              s_h).wait()

    start_ids(0, 0)

    def gpair(i, carry):
        ci = i * 2
        start_ids(ci + 1, 1)
        wait_ids(0)

        @pl.when(i > 0)
        def _():
            wait_out(0)
        compute_into(0, _CP)
        start_out(ci, 0)
        start_ids(jnp.minimum(ci + 2, nclamp), 0)
        wait_ids(1)

        @pl.when(i > 0)
        def _():
            wait_out(1)
        compute_into(1, _CP)
        start_out(ci + 1, 1)
        return carry
    lax.fori_loop(0, nfull // 2, gpair, 0)
    wait_ids(0)
    wait_out(0)
    wait_out(1)
    if tail:
        ids_v, hv, s_i, s_h = bufs[0]
        start = nfull * _CP
        pltpu.sync_copy(ids_ref.at[pl.ds(start, tail)],
                        ids_v.at[pl.ds(0, tail)])
        compute_into(0, tail)
        pltpu.sync_copy(hv.at[:, pl.ds(0, tail)],
                        out_ref.at[pl.ds(f0, 4), pl.ds(start, tail)])


def _pool_max(ht, ids, ns, *, gather_back):
    np_ = ids.shape[0]
    mesh = plsc.VectorSubcoreMesh(core_axis_name="c", subcore_axis_name="s")
    out_shape = (128, np_) if gather_back else (128, ns)
    body = functools.partial(_pool_max_body, np_=np_, ns=ns,
                             gather_back=gather_back)
    k = pl.kernel(
        body,
        out_type=jax.ShapeDtypeStruct(out_shape, jnp.float32),
        mesh=mesh,
        compiler_params=_sc_params(),
        scratch_types=[
            pltpu.VMEM((_CP,), jnp.int32),
            pltpu.VMEM((_CP,), jnp.int32),
            pltpu.VMEM((4, _CP), jnp.float32),
            pltpu.VMEM((4, _CP), jnp.float32),
            pltpu.VMEM((ns,), jnp.float32),
            pltpu.VMEM((ns,), jnp.float32),
            pltpu.VMEM((ns,), jnp.float32),
            pltpu.VMEM((ns,), jnp.float32),
            pltpu.SemaphoreType.DMA,
            pltpu.SemaphoreType.DMA,
            pltpu.SemaphoreType.DMA,
            pltpu.SemaphoreType.DMA,
        ],
    )
    return k(ht, ids)


def _pool_adds_body(ids_ref, pos_ref, out_ref, ids_v, posb, acc, *, np_, ns):
    cid = lax.axis_index("c")
    sid = lax.axis_index("s")
    wid = sid * 2 + cid
    col = wid % 4
    shard = wid // 4

    zeros = jnp.zeros((_LANES,), jnp.float32)

    def zero_body(i, carry):
        acc[pl.ds(i * _LANES, _LANES)] = zeros
        return carry
    lax.fori_loop(0, ns // _LANES, zero_body, 0)
    if ns % _LANES:
        acc[pl.ds(ns - _LANES, _LANES)] = zeros

    nchunks = np_ // _CP
    assert nchunks * _CP == np_
    iota = lax.iota(jnp.int32, _LANES)
    colsplat = jnp.full((_LANES,), 0, jnp.int32) + jnp.minimum(col, 2)
    is_cnt = col == 3
    ones = jnp.ones((_LANES,), jnp.float32)

    for k in range((nchunks + 7) // 8):
        ci = shard + 8 * k

        @pl.when(ci < nchunks)
        def _():
            start = ci * _CP
            pltpu.sync_copy(ids_ref.at[pl.ds(start, _CP)], ids_v)
            pltpu.sync_copy(pos_ref.at[pl.ds(start, _CP)], posb)

            @plsc.parallel_loop(0, _CP // _LANES, 1, unroll=4)
            def grp(g):
                c = ids_v[pl.ds(g * _LANES, _LANES)]
                rows = g * _LANES + iota
                v = plsc.load_gather(posb, [rows, colsplat])
                v = jnp.where(is_cnt, ones, v)
                plsc.addupdate_scatter(acc, [c], v)

    pltpu.sync_copy(acc, out_ref.at[col, shard])


def _pool_adds(ids, pos, ns):
    np_ = ids.shape[0]
    mesh = plsc.VectorSubcoreMesh(core_axis_name="c", subcore_axis_name="s")
    body = functools.partial(_pool_adds_body, np_=np_, ns=ns)
    k = pl.kernel(
        body,
        out_type=jax.ShapeDtypeStruct((4, 8, ns), jnp.float32),
        mesh=mesh,
        compiler_params=_sc_params(),
        scratch_types=[
            pltpu.VMEM((_CP,), jnp.int32),
            pltpu.VMEM((_CP, 3), jnp.float32),
            pltpu.VMEM((ns,), jnp.float32),
        ],
    )
    return k(ids, pos)


def _gather_cols_body(tab_ref, ids_ref, out_ref, tabv, ids0, ids1, ub0, ub1,
                      s_i0, s_i1, s_h0, s_h1, *, np_, ns):
    cid = lax.axis_index("c")
    sid = lax.axis_index("s")
    wid = sid * 2 + cid
    f0 = wid * 4
    bufs = ((ids0, ub0, s_i0, s_h0), (ids1, ub1, s_i1, s_h1))

    pltpu.sync_copy(tab_ref.at[pl.ds(f0, 4), :], tabv)

    nfull = np_ // _CP
    tail = np_ - nfull * _CP
    assert nfull % 2 == 0
    nclamp = nfull - 1

    def compute_into(b, cp):
        ids_v, ub, _, _ = bufs[b]

        def one(off):
            c = ids_v[pl.ds(off, _LANES)]
            for j in range(4):
                ub[j, pl.ds(off, _LANES)] = plsc.load_gather(tabv.at[j], [c])

        @plsc.parallel_loop(0, cp // _LANES, 1, unroll=4)
        def ploop(g):
            one(g * _LANES)
        if cp % _LANES:
            one(cp - _LANES)

    def start_ids(ci, b):
        ids_v, _, s_i, _ = bufs[b]
        pltpu.async_copy(ids_ref.at[pl.ds(ci * _CP, _CP)], ids_v, s_i)

    def wait_ids(b):
        ids_v, _, s_i, _ = bufs[b]
        pltpu.make_async_copy(ids_ref.at[pl.ds(0, _CP)], ids_v, s_i).wait()

    def start_out(ci, b):
        _, ub, _, s_h = bufs[b]
        pltpu.async_copy(ub, out_ref.at[pl.ds(f0, 4), pl.ds(ci * _CP, _CP)],
                         s_h)

    def wait_out(b):
        _, ub, _, s_h = bufs[b]
        pltpu.make_async_copy(ub, out_ref.at[pl.ds(f0, 4), pl.ds(0, _CP)],
                              s_h).wait()

    start_ids(0, 0)

    def gpair(i, carry):
        ci = i * 2
        start_ids(ci + 1, 1)
        wait_ids(0)

        @pl.when(i > 0)
        def _():
            wait_out(0)
        compute_into(0, _CP)
        start_out(ci, 0)
        start_ids(jnp.minimum(ci + 2, nclamp), 0)
        wait_ids(1)

        @pl.when(i > 0)
        def _():
            wait_out(1)
        compute_into(1, _CP)
        start_out(ci + 1, 1)
        return carry
    lax.fori_loop(0, nfull // 2, gpair, 0)
    wait_ids(0)
    wait_out(0)
    wait_out(1)
    if tail:
        ids_v, ub, s_i, s_h = bufs[0]
        start = nfull * _CP
        pltpu.sync_copy(ids_ref.at[pl.ds(start, tail)],
                        ids_v.at[pl.ds(0, tail)])
        compute_into(0, tail)
        pltpu.sync_copy(ub.at[:, pl.ds(0, tail)],
                        out_ref.at[pl.ds(f0, 4), pl.ds(start, tail)])


def _gather_cols(tab_t, ids):
    np_ = ids.shape[0]
    ns = tab_t.shape[1]
    mesh = plsc.VectorSubcoreMesh(core_axis_name="c", subcore_axis_name="s")
    body = functools.partial(_gather_cols_body, np_=np_, ns=ns)
    k = pl.kernel(
        body,
        out_type=jax.ShapeDtypeStruct((128, np_), jnp.float32),
        mesh=mesh,
        compiler_params=_sc_params(),
        scratch_types=[
            pltpu.VMEM((4, ns), jnp.float32),
            pltpu.VMEM((_CP,), jnp.int32),
            pltpu.VMEM((_CP,), jnp.int32),
            pltpu.VMEM((4, _CP), jnp.float32),
            pltpu.VMEM((4, _CP), jnp.float32),
            pltpu.SemaphoreType.DMA,
            pltpu.SemaphoreType.DMA,
            pltpu.SemaphoreType.DMA,
            pltpu.SemaphoreType.DMA,
        ],
    )
    return k(tab_t, ids)


def kernel(x, pos, cluster0, cluster1, Wp0, bp0, Wp1, bp1, Wm1, bm1, Wm0, bm0):
    D = x.shape[1]
    c0 = cluster0.astype(jnp.int32)
    c1 = cluster1.astype(jnp.int32)

    h0t = _mm(x, pos, Wp0[:D], Wp0[D:], bp0, out_transposed=True)
    x1t = _pool_max(h0t, c0, _N1, gather_back=False)
    padd = _pool_adds(c0, pos, _N1)

    h1t = _mm2(x1t, padd, Wp1[:D], Wp1[D:], bp1)
    up1t = _pool_max(h1t, c1, _N2, gather_back=True)

    h2t = _mm(up1t, x1t, Wm1[:D], Wm1[D:], bm1, a_transposed=True,
              b_transposed=True, out_transposed=True)
    up0t = _gather_cols(h2t, c0)
    out = _mm(up0t, x, Wm0[:D], Wm0[D:], bm0, a_transposed=True)
    return out
